# baseline (speedup 1.0000x reference)
import functools

import jax
import jax.numpy as jnp
from jax import lax
from jax.experimental import pallas as pl
from jax.experimental.pallas import tpu as pltpu
from jax.experimental.pallas import tpu_sc as plsc

_NC = 2
_NS = 16
_NW = _NC * _NS
_LANES = 16


def _emb_kernel(x_hbm, table_hbm, pos_hbm, out_hbm, idx_v, pos_v,
                row_a, row_b, row_c, gsem_a, gsem_b, gsem_c,
                wsem_a, wsem_b, wsem_c, psem, isem,
                *, batch, seq, d_model, p_per_w, chunk):
  cid = lax.axis_index("c")
  sid = lax.axis_index("s")
  wid = sid * _NC + cid
  p0 = pl.multiple_of(wid * p_per_w, 8)

  n_chunks = p_per_w // chunk
  g_per_row = d_model // _LANES
  rows = (row_a, row_b, row_c)
  gsems = (gsem_a, gsem_b, gsem_c)
  wsems = (wsem_a, wsem_b, wsem_c)
  nbuf = len(rows)
  steps = [(b, h) for b in range(batch) for h in range(n_chunks)]

  def start_gather(t):
    b, h = steps[t]
    idx_slice = idx_v.at[pl.ds(b * p_per_w + h * chunk, chunk)]
    return pltpu.async_copy(
        table_hbm.at[idx_slice], rows[t % nbuf], gsems[t % nbuf])

  gathers = [None] * nbuf
  writebacks = [None] * nbuf

  pltpu.sync_copy(x_hbm.at[0, pl.ds(p0, p_per_w)],
                  idx_v.at[pl.ds(0, p_per_w)])
  gathers[0] = start_gather(0)
  idx_copies = [
      pltpu.async_copy(
          x_hbm.at[b, pl.ds(p0, p_per_w)],
          idx_v.at[pl.ds(b * p_per_w, p_per_w)],
          isem,
      )
      for b in range(1, batch)
  ]
  pos_copy = pltpu.async_copy(pos_hbm.at[pl.ds(p0, p_per_w)], pos_v, psem)
  for t, (b, h) in enumerate(steps):
    cur = t % nbuf
    if t == n_chunks - 1:
      for c in idx_copies:
        c.wait()
    if t + 1 < len(steps):
      nxt = (t + 1) % nbuf
      if writebacks[nxt] is not None:
        for w in writebacks[nxt]:
          w.wait()
        writebacks[nxt] = None
      gathers[nxt] = start_gather(t + 1)
    gathers[cur].wait()
    if t == 0:
      pos_copy.wait()

    row_v = rows[cur]

    @plsc.parallel_loop(0, chunk, unroll=4)
    def add_body(r, h=h, row_v=row_v):
      for g in range(g_per_row):
        c = g * _LANES
        plsc.addupdate(
            row_v.at[r, pl.ds(c, _LANES)],
            pos_v[h * chunk + r, pl.ds(c, _LANES)],
        )

    out_row0 = pl.multiple_of(p0 + h * chunk, 8)
    writebacks[cur] = [pltpu.async_copy(
        row_v, out_hbm.at[b, pl.ds(out_row0, chunk)], wsems[cur])]

  for ws in writebacks:
    if ws is not None:
      for w in ws:
        w.wait()


def kernel(x, tok_table, pos_enc):
  batch, seq = x.shape
  vocab, d_model = tok_table.shape
  x_i32 = x if x.dtype == jnp.int32 else x.astype(jnp.int32)
  pos = pos_enc[:seq]

  p_per_w = seq // _NW
  chunk = 32

  mesh = plsc.VectorSubcoreMesh(core_axis_name="c", subcore_axis_name="s")
  k = pl.kernel(
      functools.partial(
          _emb_kernel,
          batch=batch, seq=seq, d_model=d_model,
          p_per_w=p_per_w, chunk=chunk,
      ),
      out_type=jax.ShapeDtypeStruct((batch, seq, d_model), jnp.float32),
      mesh=mesh,
      scratch_types=[
          pltpu.VMEM((batch * p_per_w,), jnp.int32),
          pltpu.VMEM((p_per_w, d_model), jnp.float32),
          pltpu.VMEM((chunk, d_model), jnp.float32),
          pltpu.VMEM((chunk, d_model), jnp.float32),
          pltpu.VMEM((chunk, d_model), jnp.float32),
          pltpu.SemaphoreType.DMA,
          pltpu.SemaphoreType.DMA,
          pltpu.SemaphoreType.DMA,
          pltpu.SemaphoreType.DMA,
          pltpu.SemaphoreType.DMA,
          pltpu.SemaphoreType.DMA,
          pltpu.SemaphoreType.DMA,
          pltpu.SemaphoreType.DMA,
      ],
  )
  return k(x_i32, tok_table, pos)


# speedup vs baseline: 1.1386x; 1.1386x over previous
import functools

import jax
import jax.numpy as jnp
from jax import lax
from jax.experimental import pallas as pl
from jax.experimental.pallas import tpu as pltpu
from jax.experimental.pallas import tpu_sc as plsc

_NC = 2
_NS = 16
_NW = _NC * _NS
_LANES = 16


def _emb_kernel(x_hbm, table_hbm, pos_hbm, out_hbm, idx_v, pos_v,
                row_a, row_b, row_c, gsem_a, gsem_b, gsem_c,
                wsem_a, wsem_b, wsem_c, psem, isem,
                *, batch, seq, d_model, p_per_w, chunk):
  cid = lax.axis_index("c")
  sid = lax.axis_index("s")
  wid = sid * _NC + cid
  p0 = pl.multiple_of(wid * p_per_w, 8)

  n_chunks = p_per_w // chunk
  g_per_row = d_model // _LANES
  rows = (row_a, row_b, row_c)
  gsems = (gsem_a, gsem_b, gsem_c)
  wsems = (wsem_a, wsem_b, wsem_c)
  nbuf = len(rows)
  steps = [(b, h) for b in range(batch) for h in range(n_chunks)]

  def start_gather(t):
    b, h = steps[t]
    idx_slice = idx_v.at[pl.ds(b * p_per_w + h * chunk, chunk)]
    return pltpu.async_copy(
        table_hbm.at[idx_slice], rows[t % nbuf], gsems[t % nbuf])

  gathers = [None] * nbuf
  writebacks = [None] * nbuf

  pltpu.sync_copy(x_hbm.at[0, pl.ds(p0, p_per_w)],
                  idx_v.at[pl.ds(0, p_per_w)])
  gathers[0] = start_gather(0)
  idx_copies = [
      pltpu.async_copy(
          x_hbm.at[b, pl.ds(p0, p_per_w)],
          idx_v.at[pl.ds(b * p_per_w, p_per_w)],
          isem,
      )
      for b in range(1, batch)
  ]
  pos_copy = pltpu.async_copy(pos_hbm.at[pl.ds(p0, p_per_w)], pos_v, psem)
  for t, (b, h) in enumerate(steps):
    cur = t % nbuf
    if t == n_chunks - 1:
      for c in idx_copies:
        c.wait()
    if t + 1 < len(steps):
      nxt = (t + 1) % nbuf
      if writebacks[nxt] is not None:
        for w in writebacks[nxt]:
          w.wait()
        writebacks[nxt] = None
      gathers[nxt] = start_gather(t + 1)
    gathers[cur].wait()
    if t == 0:
      pos_copy.wait()

    row_v = rows[cur]

    @plsc.parallel_loop(0, chunk)
    def add_body(r, h=h, row_v=row_v):
      for g in range(g_per_row):
        c = g * _LANES
        plsc.addupdate(
            row_v.at[r, pl.ds(c, _LANES)],
            pos_v[h * chunk + r, pl.ds(c, _LANES)],
        )

    out_row0 = pl.multiple_of(p0 + h * chunk, 8)
    writebacks[cur] = [pltpu.async_copy(
        row_v, out_hbm.at[b, pl.ds(out_row0, chunk)], wsems[cur])]

  for ws in writebacks:
    if ws is not None:
      for w in ws:
        w.wait()


def kernel(x, tok_table, pos_enc):
  batch, seq = x.shape
  vocab, d_model = tok_table.shape
  x_i32 = x if x.dtype == jnp.int32 else x.astype(jnp.int32)
  pos = pos_enc[:seq]

  p_per_w = seq // _NW
  chunk = 32

  mesh = plsc.VectorSubcoreMesh(core_axis_name="c", subcore_axis_name="s")
  k = pl.kernel(
      functools.partial(
          _emb_kernel,
          batch=batch, seq=seq, d_model=d_model,
          p_per_w=p_per_w, chunk=chunk,
      ),
      out_type=jax.ShapeDtypeStruct((batch, seq, d_model), jnp.float32),
      mesh=mesh,
      scratch_types=[
          pltpu.VMEM((batch * p_per_w,), jnp.int32),
          pltpu.VMEM((p_per_w, d_model), jnp.float32),
          pltpu.VMEM((chunk, d_model), jnp.float32),
          pltpu.VMEM((chunk, d_model), jnp.float32),
          pltpu.VMEM((chunk, d_model), jnp.float32),
          pltpu.SemaphoreType.DMA,
          pltpu.SemaphoreType.DMA,
          pltpu.SemaphoreType.DMA,
          pltpu.SemaphoreType.DMA,
          pltpu.SemaphoreType.DMA,
          pltpu.SemaphoreType.DMA,
          pltpu.SemaphoreType.DMA,
          pltpu.SemaphoreType.DMA,
      ],
  )
  return k(x_i32, tok_table, pos)


# speedup vs baseline: 1.1572x; 1.0163x over previous
import functools

import jax
import jax.numpy as jnp
from jax import lax
from jax.experimental import pallas as pl
from jax.experimental.pallas import tpu as pltpu
from jax.experimental.pallas import tpu_sc as plsc

_NC = 2
_NS = 16
_NW = _NC * _NS
_LANES = 16


def _emb_kernel(x_hbm, table_hbm, pos_hbm, out_hbm, idx_v, pos_v,
                row_a, row_b, row_c, gsem_a, gsem_b, gsem_c,
                wsem_a, wsem_b, wsem_c, psem, isem,
                *, batch, seq, d_model, p_per_w, chunk):
  cid = lax.axis_index("c")
  sid = lax.axis_index("s")
  wid = sid * _NC + cid
  p0 = pl.multiple_of(wid * p_per_w, 8)

  n_chunks = p_per_w // chunk
  g_per_row = d_model // _LANES
  rows = (row_a, row_b, row_c)
  gsems = (gsem_a, gsem_b, gsem_c)
  wsems = (wsem_a, wsem_b, wsem_c)
  nbuf = len(rows)
  steps = [(b, h) for b in range(batch) for h in range(n_chunks)]

  def start_gather(t):
    b, h = steps[t]
    idx_slice = idx_v.at[pl.ds(b * p_per_w + h * chunk, chunk)]
    return pltpu.async_copy(
        table_hbm.at[idx_slice], rows[t % nbuf], gsems[t % nbuf])

  gathers = [None] * nbuf
  writebacks = [None] * nbuf

  pltpu.sync_copy(x_hbm.at[0, pl.ds(p0, p_per_w)],
                  idx_v.at[pl.ds(0, p_per_w)])
  gathers[0] = start_gather(0)
  idx_copies = [
      pltpu.async_copy(
          x_hbm.at[b, pl.ds(p0, p_per_w)],
          idx_v.at[pl.ds(b * p_per_w, p_per_w)],
          isem,
      )
      for b in range(1, batch)
  ]
  half_p = p_per_w // 2
  pos_copies = [
      pltpu.async_copy(
          pos_hbm.at[pl.ds(p0 + i * half_p, half_p)],
          pos_v.at[pl.ds(i * half_p, half_p)],
          psem,
      )
      for i in range(2)
  ]
  for t, (b, h) in enumerate(steps):
    cur = t % nbuf
    if t == n_chunks - 1:
      for c in idx_copies:
        c.wait()
    if t + 1 < len(steps):
      nxt = (t + 1) % nbuf
      if writebacks[nxt] is not None:
        for w in writebacks[nxt]:
          w.wait()
        writebacks[nxt] = None
      gathers[nxt] = start_gather(t + 1)
    gathers[cur].wait()
    if t < 2 and pos_copies[h] is not None:
      pos_copies[h].wait()
      pos_copies[h] = None

    row_v = rows[cur]

    @plsc.parallel_loop(0, chunk)
    def add_body(r, h=h, row_v=row_v):
      for g in range(g_per_row):
        c = g * _LANES
        plsc.addupdate(
            row_v.at[r, pl.ds(c, _LANES)],
            pos_v[h * chunk + r, pl.ds(c, _LANES)],
        )

    out_row0 = pl.multiple_of(p0 + h * chunk, 8)
    writebacks[cur] = [pltpu.async_copy(
        row_v, out_hbm.at[b, pl.ds(out_row0, chunk)], wsems[cur])]

  for ws in writebacks:
    if ws is not None:
      for w in ws:
        w.wait()


def kernel(x, tok_table, pos_enc):
  batch, seq = x.shape
  vocab, d_model = tok_table.shape
  x_i32 = x if x.dtype == jnp.int32 else x.astype(jnp.int32)
  pos = pos_enc[:seq]

  p_per_w = seq // _NW
  chunk = 32

  mesh = plsc.VectorSubcoreMesh(core_axis_name="c", subcore_axis_name="s")
  k = pl.kernel(
      functools.partial(
          _emb_kernel,
          batch=batch, seq=seq, d_model=d_model,
          p_per_w=p_per_w, chunk=chunk,
      ),
      out_type=jax.ShapeDtypeStruct((batch, seq, d_model), jnp.float32),
      mesh=mesh,
      scratch_types=[
          pltpu.VMEM((batch * p_per_w,), jnp.int32),
          pltpu.VMEM((p_per_w, d_model), jnp.float32),
          pltpu.VMEM((chunk, d_model), jnp.float32),
          pltpu.VMEM((chunk, d_model), jnp.float32),
          pltpu.VMEM((chunk, d_model), jnp.float32),
          pltpu.SemaphoreType.DMA,
          pltpu.SemaphoreType.DMA,
          pltpu.SemaphoreType.DMA,
          pltpu.SemaphoreType.DMA,
          pltpu.SemaphoreType.DMA,
          pltpu.SemaphoreType.DMA,
          pltpu.SemaphoreType.DMA,
          pltpu.SemaphoreType.DMA,
      ],
  )
  return k(x_i32, tok_table, pos)


# speedup vs baseline: 1.1572x; 1.0000x over previous
import functools

import jax
import jax.numpy as jnp
from jax import lax
from jax.experimental import pallas as pl
from jax.experimental.pallas import tpu as pltpu
from jax.experimental.pallas import tpu_sc as plsc

_NC = 2
_NS = 16
_NW = _NC * _NS
_LANES = 16


def _emb_kernel(x_hbm, table_hbm, pos_hbm, out_hbm, idx_v, pos_v,
                row_a, row_b, row_c, gsem_a, gsem_b, gsem_c,
                wsem_a, wsem_b, wsem_c, psem, isem,
                *, batch, seq, d_model, p_per_w, chunk):
  cid = lax.axis_index("c")
  sid = lax.axis_index("s")
  wid = sid * _NC + cid
  p0 = pl.multiple_of(wid * p_per_w, 8)

  n_chunks = p_per_w // chunk
  g_per_row = d_model // _LANES
  rows = (row_a, row_b, row_c)
  gsems = (gsem_a, gsem_b, gsem_c)
  wsems = (wsem_a, wsem_b, wsem_c)
  nbuf = len(rows)
  steps = [(b, h) for b in range(batch) for h in range(n_chunks)]

  def start_gather(t):
    b, h = steps[t]
    idx_slice = idx_v.at[pl.ds(b * p_per_w + h * chunk, chunk)]
    return pltpu.async_copy(
        table_hbm.at[idx_slice], rows[t % nbuf], gsems[t % nbuf])

  gathers = [None] * nbuf
  writebacks = [None] * nbuf

  pltpu.sync_copy(x_hbm.at[0, pl.ds(p0, p_per_w)],
                  idx_v.at[pl.ds(0, p_per_w)])
  gathers[0] = start_gather(0)
  idx_copies = [
      pltpu.async_copy(
          x_hbm.at[b, pl.ds(p0, p_per_w)],
          idx_v.at[pl.ds(b * p_per_w, p_per_w)],
          isem,
      )
      for b in range(1, batch)
  ]
  pos_copies = [
      pltpu.async_copy(
          pos_hbm.at[pl.ds(p0 + i * chunk, chunk)],
          pos_v.at[pl.ds(i * chunk, chunk)],
          psem,
      )
      for i in range(n_chunks)
  ]
  for t, (b, h) in enumerate(steps):
    cur = t % nbuf
    if t == n_chunks - 1:
      for c in idx_copies:
        c.wait()
    if t + 1 < len(steps):
      nxt = (t + 1) % nbuf
      if writebacks[nxt] is not None:
        for w in writebacks[nxt]:
          w.wait()
        writebacks[nxt] = None
      gathers[nxt] = start_gather(t + 1)
    gathers[cur].wait()
    if t < n_chunks and pos_copies[h] is not None:
      pos_copies[h].wait()
      pos_copies[h] = None

    row_v = rows[cur]

    @plsc.parallel_loop(0, chunk)
    def add_body(r, h=h, row_v=row_v):
      for g in range(g_per_row):
        c = g * _LANES
        plsc.addupdate(
            row_v.at[r, pl.ds(c, _LANES)],
            pos_v[h * chunk + r, pl.ds(c, _LANES)],
        )

    out_row0 = pl.multiple_of(p0 + h * chunk, 8)
    writebacks[cur] = [pltpu.async_copy(
        row_v, out_hbm.at[b, pl.ds(out_row0, chunk)], wsems[cur])]

  for ws in writebacks:
    if ws is not None:
      for w in ws:
        w.wait()


def kernel(x, tok_table, pos_enc):
  batch, seq = x.shape
  vocab, d_model = tok_table.shape
  x_i32 = x if x.dtype == jnp.int32 else x.astype(jnp.int32)
  pos = pos_enc[:seq]

  p_per_w = seq // _NW
  chunk = 32

  mesh = plsc.VectorSubcoreMesh(core_axis_name="c", subcore_axis_name="s")
  k = pl.kernel(
      functools.partial(
          _emb_kernel,
          batch=batch, seq=seq, d_model=d_model,
          p_per_w=p_per_w, chunk=chunk,
      ),
      out_type=jax.ShapeDtypeStruct((batch, seq, d_model), jnp.float32),
      mesh=mesh,
      scratch_types=[
          pltpu.VMEM((batch * p_per_w,), jnp.int32),
          pltpu.VMEM((p_per_w, d_model), jnp.float32),
          pltpu.VMEM((chunk, d_model), jnp.float32),
          pltpu.VMEM((chunk, d_model), jnp.float32),
          pltpu.VMEM((chunk, d_model), jnp.float32),
          pltpu.SemaphoreType.DMA,
          pltpu.SemaphoreType.DMA,
          pltpu.SemaphoreType.DMA,
          pltpu.SemaphoreType.DMA,
          pltpu.SemaphoreType.DMA,
          pltpu.SemaphoreType.DMA,
          pltpu.SemaphoreType.DMA,
          pltpu.SemaphoreType.DMA,
      ],
  )
  return k(x_i32, tok_table, pos)
